# Initial kernel scaffold; baseline (speedup 1.0000x reference)
import jax
import jax.numpy as jnp
from jax.experimental import pallas as pl


def kernel(bodys, heads, emb, Wih, Whh, bih, bhh, fc_w, fc_b, fck_w, fck_b, fcq_w, fcq_b):
    raise NotImplementedError("write your pallas kernel here")


# speedup vs baseline: 8.1067x; 8.1067x over previous
import functools

import jax
import jax.numpy as jnp
from jax.experimental import pallas as pl

_R = 1000
_E = 64
_B = 1024
_BB = 256
_HIGH = jax.lax.Precision.HIGHEST


def _fused(bodys_ref, heads_ref, emb_ref, embT_ref,
           wi_ref, wh_ref, bih_ref, bhh_ref,
           fcw_ref, fck_ref, fckT_ref, fckb_ref, fckbT_ref,
           fcqT_ref, fcqb_ref,
           pred_rel_ref, pred_last_ref, loss_ref):
    emb = emb_ref[...]
    iota_r = jax.lax.broadcasted_iota(jnp.int32, (_BB, _R), 1)

    def gather(idx_col):
        onehot = (iota_r == idx_col).astype(jnp.float32)
        return jnp.dot(onehot, emb, precision=_HIGH)

    x0 = gather(bodys_ref[:, 0:1])
    x1 = gather(bodys_ref[:, 1:2])
    x2 = gather(bodys_ref[:, 2:3])

    def xw(x):
        return [jnp.dot(x, wi_ref[k]) for k in range(4)]

    def lstm2(xw1, xw2):
        g = [xw1[k] + bih_ref[k:k + 1, :] + bhh_ref[k:k + 1, :] for k in range(4)]
        c = jax.nn.sigmoid(g[0]) * jnp.tanh(g[2])
        h = jax.nn.sigmoid(g[3]) * jnp.tanh(c)
        g = [xw2[k] + bih_ref[k:k + 1, :] + jnp.dot(h, wh_ref[k])
             + bhh_ref[k:k + 1, :] for k in range(4)]
        c = jax.nn.sigmoid(g[1]) * c + jax.nn.sigmoid(g[0]) * jnp.tanh(g[2])
        h = jax.nn.sigmoid(g[3]) * jnp.tanh(c)
        return h

    xw0, xw1_, xw2_ = xw(x0), xw(x1), xw(x2)
    h_p0 = lstm2(xw0, xw1_)
    h_p1 = lstm2(xw1_, xw2_)

    fcw = fcw_ref[...]
    p0 = jnp.sum(h_p0 * fcw, axis=1, keepdims=True)
    p1 = jnp.sum(h_p1 * fcw, axis=1, keepdims=True)
    sel0 = p0 >= p1
    sel_h = jnp.where(sel0, h_p0, h_p1)

    key_relT = jnp.dot(fck_ref[...], embT_ref[...]) + fckbT_ref[...]
    q = jnp.dot(sel_h, fcqT_ref[...]) + fcqb_ref[...]
    key_sel = jnp.dot(sel_h, fckT_ref[...]) + fckb_ref[...]
    s_rel = jnp.dot(q, key_relT) / 8.0
    s_last = jnp.sum(q * key_sel, axis=1, keepdims=True) / 8.0
    m = jnp.maximum(jnp.max(s_rel, axis=1, keepdims=True), s_last)
    e_rel = jnp.exp(s_rel - m)
    e_last = jnp.exp(s_last - m)
    den = jnp.sum(e_rel, axis=1, keepdims=True) + e_last
    merged = jnp.dot(e_rel / den, emb) + (e_last / den) * sel_h

    row0 = jnp.where(sel0, merged, x0)
    row1 = jnp.where(sel0, x2, merged)

    h = lstm2(xw(row0), xw(row1))

    q2 = jnp.dot(h, fcqT_ref[...]) + fcqb_ref[...]
    key_h = jnp.dot(h, fckT_ref[...]) + fckb_ref[...]
    pred_rel = jnp.dot(q2, key_relT) / 8.0
    pred_last = jnp.sum(q2 * key_h, axis=1, keepdims=True) / 8.0
    pred_rel_ref[...] = pred_rel
    pred_last_ref[...] = pred_last

    m2 = jnp.maximum(jnp.max(pred_rel, axis=1, keepdims=True), pred_last)
    lse = jnp.log(jnp.sum(jnp.exp(pred_rel - m2), axis=1, keepdims=True)
                  + jnp.exp(pred_last - m2)) + m2
    picked = jnp.sum(jnp.where(iota_r == heads_ref[...], pred_rel, 0.0),
                     axis=1, keepdims=True)
    part = jnp.sum(lse - picked, keepdims=True)

    @pl.when(pl.program_id(0) == 0)
    def _init():
        loss_ref[...] = jnp.zeros_like(loss_ref)

    loss_ref[...] += part


@functools.partial(jax.jit, static_argnums=())
def kernel(bodys, heads, emb, Wih, Whh, bih, bhh, fc_w, fc_b,
           fck_w, fck_b, fcq_w, fcq_b):
    emb_rel = emb[:_R]
    embT = emb_rel.T
    wi = jnp.stack([Wih[k * _E:(k + 1) * _E].T for k in range(4)])
    wh = jnp.stack([Whh[k * _E:(k + 1) * _E].T for k in range(4)])
    bih4 = bih.reshape(4, _E)
    bhh4 = bhh.reshape(4, _E)

    grid = _B // _BB
    blk = lambda *shape: pl.BlockSpec(shape, lambda i: (0,) * len(shape))
    pred_rel, pred_last, loss = pl.pallas_call(
        _fused,
        grid=(grid,),
        in_specs=[
            pl.BlockSpec((_BB, 3), lambda i: (i, 0)),
            pl.BlockSpec((_BB, 1), lambda i: (i, 0)),
            blk(_R, _E),
            blk(_E, _R),
            blk(4, _E, _E),
            blk(4, _E, _E),
            blk(4, _E),
            blk(4, _E),
            blk(1, _E),
            blk(_E, _E),
            blk(_E, _E),
            blk(1, _E),
            blk(_E, 1),
            blk(_E, _E),
            blk(1, _E),
        ],
        out_specs=[
            pl.BlockSpec((_BB, _R), lambda i: (i, 0)),
            pl.BlockSpec((_BB, 1), lambda i: (i, 0)),
            pl.BlockSpec((1, 1), lambda i: (0, 0)),
        ],
        out_shape=[
            jax.ShapeDtypeStruct((_B, _R), jnp.float32),
            jax.ShapeDtypeStruct((_B, 1), jnp.float32),
            jax.ShapeDtypeStruct((1, 1), jnp.float32),
        ],
    )(bodys.astype(jnp.int32), heads.astype(jnp.int32).reshape(_B, 1),
      emb_rel, embT, wi, wh, bih4, bhh4,
      fc_w, fck_w, fck_w.T, fck_b.reshape(1, _E), fck_b.reshape(_E, 1),
      fcq_w.T, fcq_b.reshape(1, _E))

    pred = jnp.concatenate([pred_rel, pred_last], axis=1)
    loss = loss[0, 0] / _B
    return (pred, loss)


# speedup vs baseline: 9.7255x; 1.1997x over previous
import jax
import jax.numpy as jnp
from jax.experimental import pallas as pl

_R = 1000
_E = 64
_B = 1024
_BB = 1024
_HIGH = jax.lax.Precision.HIGHEST
_NT = (((1,), (1,)), ((), ()))


def _fused(bodys_ref, heads_ref, emb_ref, wih_ref, whh_ref, bih_ref, bhh_ref,
           fcw_ref, fck_ref, fckb_ref, fckbT_ref, fcq_ref, fcqb_ref,
           pred_ref, loss_ref):
    emb = emb_ref[0:_R, :]
    iota_r = jax.lax.broadcasted_iota(jnp.int32, (_BB, _R), 1)

    def gather(idx_col):
        onehot = (iota_r == idx_col).astype(jnp.float32)
        return jnp.dot(onehot, emb, precision=_HIGH)

    x0 = gather(bodys_ref[:, 0:1])
    x1 = gather(bodys_ref[:, 1:2])
    x2 = gather(bodys_ref[:, 2:3])

    def xw(x):
        return [jax.lax.dot_general(x, wih_ref[k * _E:(k + 1) * _E, :], _NT)
                for k in range(4)]

    def lstm2(xw1, xw2):
        g = [xw1[k] + bih_ref[k:k + 1, :] + bhh_ref[k:k + 1, :] for k in range(4)]
        c = jax.nn.sigmoid(g[0]) * jnp.tanh(g[2])
        h = jax.nn.sigmoid(g[3]) * jnp.tanh(c)
        g = [xw2[k] + bih_ref[k:k + 1, :]
             + jax.lax.dot_general(h, whh_ref[k * _E:(k + 1) * _E, :], _NT)
             + bhh_ref[k:k + 1, :] for k in range(4)]
        c = jax.nn.sigmoid(g[1]) * c + jax.nn.sigmoid(g[0]) * jnp.tanh(g[2])
        h = jax.nn.sigmoid(g[3]) * jnp.tanh(c)
        return h

    xw0, xw1_, xw2_ = xw(x0), xw(x1), xw(x2)
    h_p0 = lstm2(xw0, xw1_)
    h_p1 = lstm2(xw1_, xw2_)

    fcw = fcw_ref[...]
    p0 = jnp.sum(h_p0 * fcw, axis=1, keepdims=True)
    p1 = jnp.sum(h_p1 * fcw, axis=1, keepdims=True)
    sel0 = p0 >= p1
    sel_h = jnp.where(sel0, h_p0, h_p1)

    key_relT = jax.lax.dot_general(fck_ref[...], emb, _NT) + fckbT_ref[...]
    q = jax.lax.dot_general(sel_h, fcq_ref[...], _NT) + fcqb_ref[...]
    key_sel = jax.lax.dot_general(sel_h, fck_ref[...], _NT) + fckb_ref[...]
    s_rel = jnp.dot(q, key_relT) / 8.0
    s_last = jnp.sum(q * key_sel, axis=1, keepdims=True) / 8.0
    m = jnp.maximum(jnp.max(s_rel, axis=1, keepdims=True), s_last)
    e_rel = jnp.exp(s_rel - m)
    e_last = jnp.exp(s_last - m)
    den = jnp.sum(e_rel, axis=1, keepdims=True) + e_last
    merged = jnp.dot(e_rel / den, emb) + (e_last / den) * sel_h

    row0 = jnp.where(sel0, merged, x0)
    row1 = jnp.where(sel0, x2, merged)

    h = lstm2(xw(row0), xw(row1))

    q2 = jax.lax.dot_general(h, fcq_ref[...], _NT) + fcqb_ref[...]
    key_h = jax.lax.dot_general(h, fck_ref[...], _NT) + fckb_ref[...]
    pred_rel = jnp.dot(q2, key_relT) / 8.0
    pred_last = jnp.sum(q2 * key_h, axis=1, keepdims=True) / 8.0
    pred_ref[...] = jnp.concatenate([pred_rel, pred_last], axis=1)

    m2 = jnp.maximum(jnp.max(pred_rel, axis=1, keepdims=True), pred_last)
    lse = jnp.log(jnp.sum(jnp.exp(pred_rel - m2), axis=1, keepdims=True)
                  + jnp.exp(pred_last - m2)) + m2
    picked = jnp.sum(jnp.where(iota_r == heads_ref[...], pred_rel, 0.0),
                     axis=1, keepdims=True)
    part = jnp.sum(lse - picked, keepdims=True)

    i = pl.program_id(0)

    @pl.when(i == 0)
    def _init():
        loss_ref[...] = jnp.zeros_like(loss_ref)

    loss_ref[...] += part

    @pl.when(i == _B // _BB - 1)
    def _scale():
        loss_ref[...] *= (1.0 / _B)


def kernel(bodys, heads, emb, Wih, Whh, bih, bhh, fc_w, fc_b,
           fck_w, fck_b, fcq_w, fcq_b):
    grid = _B // _BB
    blk = lambda *shape: pl.BlockSpec(shape, lambda i: (0,) * len(shape))
    pred, loss = pl.pallas_call(
        _fused,
        grid=(grid,),
        in_specs=[
            pl.BlockSpec((_BB, 3), lambda i: (i, 0)),
            pl.BlockSpec((_BB, 1), lambda i: (i, 0)),
            blk(_R + 1, _E),
            blk(4 * _E, _E),
            blk(4 * _E, _E),
            blk(4, _E),
            blk(4, _E),
            blk(1, _E),
            blk(_E, _E),
            blk(1, _E),
            blk(_E, 1),
            blk(_E, _E),
            blk(1, _E),
        ],
        out_specs=[
            pl.BlockSpec((_BB, _R + 1), lambda i: (i, 0)),
            pl.BlockSpec((1, 1), lambda i: (0, 0)),
        ],
        out_shape=[
            jax.ShapeDtypeStruct((_B, _R + 1), jnp.float32),
            jax.ShapeDtypeStruct((1, 1), jnp.float32),
        ],
    )(bodys.astype(jnp.int32), heads.astype(jnp.int32).reshape(_B, 1),
      emb, Wih, Whh, bih.reshape(4, _E), bhh.reshape(4, _E),
      fc_w, fck_w, fck_b.reshape(1, _E), fck_b.reshape(_E, 1),
      fcq_w, fcq_b.reshape(1, _E))
    return (pred, loss.reshape(()))
